# baseline (speedup 1.0000x reference)
import jax
import jax.numpy as jnp
from jax.experimental import pallas as pl
from jax.experimental.pallas import tpu as pltpu

_DOT_DIMS = (((1,), (0,)), ((), ()))
_HB = 64


def _head_kernel(x_ref, w_ref, b_ref, cls_ref, reg_ref, dir_ref):
    j = pl.program_id(1)
    HB, W, C = x_ref.shape[1], x_ref.shape[2], x_ref.shape[3]
    H = cls_ref.shape[3]
    nj = pl.num_programs(1)
    wt = w_ref[...]
    bias = b_ref[...]
    r = jax.lax.dot_general(
        x_ref[0].reshape(HB * W, C), wt, _DOT_DIMS,
        preferred_element_type=jnp.float32) + bias
    r3 = r.reshape(HB, W, 20)
    for w in range(W):
        rt = r3[:, w, :].T
        for k in range(nj):
            lo = k * HB
            span = min(HB, H - lo)

            @pl.when(j == k)
            def _(rt=rt, lo=lo, span=span):
                cls_ref[0, :, w, lo:lo + span] = rt[0:2, :span]
                reg_ref[0, :, w, lo:lo + span] = rt[2:16, :span]
                dir_ref[0, :, w, lo:lo + span] = rt[16:20, :span]


def kernel(x, W_cls, b_cls, W_reg, b_reg, W_dir, b_dir):
    B, C, H, W = x.shape
    G = pl.cdiv(H, _HB)
    oc, og, od = W_cls.shape[0], W_reg.shape[0], W_dir.shape[0]
    xt = jnp.transpose(x, (0, 2, 3, 1))
    wall = jnp.concatenate([W_cls, W_reg, W_dir], axis=0).T
    ball = jnp.concatenate([b_cls, b_reg, b_dir]).reshape(1, -1)
    no = wall.shape[1]

    def ospec(o):
        return pl.BlockSpec((1, o, W, H), lambda b, j: (b, 0, 0, 0))

    outs = pl.pallas_call(
        _head_kernel,
        grid=(B, G),
        in_specs=[
            pl.BlockSpec((1, _HB, W, C), lambda b, j: (b, j, 0, 0)),
            pl.BlockSpec((C, no), lambda b, j: (0, 0)),
            pl.BlockSpec((1, no), lambda b, j: (0, 0)),
        ],
        out_specs=[ospec(oc), ospec(og), ospec(od)],
        out_shape=[
            jax.ShapeDtypeStruct((B, oc, W, H), x.dtype),
            jax.ShapeDtypeStruct((B, og, W, H), x.dtype),
            jax.ShapeDtypeStruct((B, od, W, H), x.dtype),
        ],
        compiler_params=pltpu.CompilerParams(
            dimension_semantics=("parallel", "arbitrary")),
    )(xt, wall, ball)
    return tuple(o.transpose(0, 1, 3, 2) for o in outs)


# speedup vs baseline: 3.8599x; 3.8599x over previous
import jax
import jax.numpy as jnp
from jax.experimental import pallas as pl
from jax.experimental.pallas import tpu as pltpu

_DOT_DIMS = (((1,), (0,)), ((), ()))
_WB = 72


def _head_kernel(x_ref, w_ref, b_ref, cls_ref, reg_ref, dir_ref):
    wt = w_ref[...]
    bias = b_ref[...]
    for w in range(_WB):
        xw = x_ref[0, :, w, :]
        r = jax.lax.dot_general(
            xw, wt, _DOT_DIMS, preferred_element_type=jnp.float32) + bias
        rt = r.T
        cls_ref[0, :, w, :] = rt[0:2]
        reg_ref[0, :, w, :] = rt[2:16]
        dir_ref[0, :, w, :] = rt[16:20]


def kernel(x, W_cls, b_cls, W_reg, b_reg, W_dir, b_dir):
    B, C, H, W = x.shape
    G = pl.cdiv(W, _WB)
    oc, og, od = W_cls.shape[0], W_reg.shape[0], W_dir.shape[0]
    xt = jnp.transpose(x, (0, 2, 3, 1))
    wall = jnp.concatenate([W_cls, W_reg, W_dir], axis=0).T
    ball = jnp.concatenate([b_cls, b_reg, b_dir]).reshape(1, -1)
    no = wall.shape[1]

    def ospec(o):
        return pl.BlockSpec((1, o, _WB, H), lambda b, j: (b, 0, j, 0))

    outs = pl.pallas_call(
        _head_kernel,
        grid=(B, G),
        in_specs=[
            pl.BlockSpec((1, H, _WB, C), lambda b, j: (b, 0, j, 0)),
            pl.BlockSpec((C, no), lambda b, j: (0, 0)),
            pl.BlockSpec((1, no), lambda b, j: (0, 0)),
        ],
        out_specs=[ospec(oc), ospec(og), ospec(od)],
        out_shape=[
            jax.ShapeDtypeStruct((B, oc, W, H), x.dtype),
            jax.ShapeDtypeStruct((B, og, W, H), x.dtype),
            jax.ShapeDtypeStruct((B, od, W, H), x.dtype),
        ],
        compiler_params=pltpu.CompilerParams(
            dimension_semantics=("parallel", "parallel")),
    )(xt, wall, ball)
    return tuple(o.transpose(0, 1, 3, 2) for o in outs)
